# baseline (speedup 1.0000x reference)
import functools

import jax
import jax.numpy as jnp
from jax.experimental import pallas as pl
from jax.experimental.pallas import tpu as pltpu

ALPHA = 0.01


def _bl_kernel(x_ref, loss_ref, counts_ref, sums_ref, *, L, NB, K):
    i = pl.program_id(0)
    l = i // NB

    @pl.when(i == 0)
    def _init():
        counts_ref[...] = jnp.zeros_like(counts_ref)
        sums_ref[...] = jnp.zeros_like(sums_ref)

    x_orig = x_ref[...]
    sums_ref[pl.ds(l, 1), :] += jnp.sum(x_orig, axis=0, keepdims=True)

    x = x_orig.T
    thresh = jnp.max(x, axis=0, keepdims=True)
    for _ in range(K - 1):
        thresh = jnp.max(jnp.where(x < thresh, x, -jnp.inf), axis=0, keepdims=True)
    sel = (x >= thresh).astype(jnp.float32)
    counts_ref[pl.ds(l, 1), :] += jnp.sum(sel, axis=1, keepdims=True).T

    @pl.when(i == L * NB - 1)
    def _fin():
        loss_ref[...] = jnp.sum(counts_ref[...] * sums_ref[...]).reshape(1, 1)


def kernel(router_weights, n_routed_experts, num_experts_per_tok, router_n_groups):
    L, S, E = router_weights.shape
    K = 8
    T = min(8192, S)
    NB = S // T
    CH = 4 if L % 4 == 0 else 1
    LC = L // CH

    partials = []
    for c in range(CH):
        sl = jax.lax.slice_in_dim(router_weights, c * LC, (c + 1) * LC, axis=0)
        rwc = sl.reshape(LC * S, E)
        out = pl.pallas_call(
            functools.partial(_bl_kernel, L=LC, NB=NB, K=K),
            grid=(LC * NB,),
            in_specs=[pl.BlockSpec((T, E), lambda i: (i, 0))],
            out_specs=pl.BlockSpec((1, 1), lambda i: (0, 0)),
            out_shape=jax.ShapeDtypeStruct((1, 1), jnp.float32),
            scratch_shapes=[
                pltpu.VMEM((LC, E), jnp.float32),
                pltpu.VMEM((LC, E), jnp.float32),
            ],
        )(rwc)
        partials.append(out[0, 0])

    total = functools.reduce(lambda a, b: a + b, partials)
    scale = n_routed_experts / (S * num_experts_per_tok)
    return total * scale * (ALPHA / S)


# speedup vs baseline: 2.0843x; 2.0843x over previous
import functools

import jax
import jax.numpy as jnp
from jax.experimental import pallas as pl
from jax.experimental.pallas import tpu as pltpu

ALPHA = 0.01


def _bl_kernel(x_ref, loss_ref, counts_ref, sums_ref, *, L, NS, K):
    l = pl.program_id(0)
    s = pl.program_id(1)

    @pl.when(jnp.logical_and(l == 0, s == 0))
    def _init():
        counts_ref[...] = jnp.zeros_like(counts_ref)
        sums_ref[...] = jnp.zeros_like(sums_ref)

    x = x_ref[0]
    sums_ref[pl.ds(l, 1), :] += jnp.sum(x, axis=1, keepdims=True).T

    thresh = jnp.max(x, axis=0, keepdims=True)
    for _ in range(K - 1):
        thresh = jnp.max(jnp.where(x < thresh, x, -jnp.inf), axis=0, keepdims=True)
    sel = (x >= thresh).astype(jnp.float32)
    counts_ref[pl.ds(l, 1), :] += jnp.sum(sel, axis=1, keepdims=True).T

    @pl.when(jnp.logical_and(l == L - 1, s == NS - 1))
    def _fin():
        loss_ref[...] = jnp.sum(counts_ref[...] * sums_ref[...]).reshape(1, 1)


def kernel(router_weights, n_routed_experts, num_experts_per_tok, router_n_groups):
    L, S, E = router_weights.shape
    K = 8
    T = min(8192, S)
    NS = S // T
    rwt = jnp.transpose(router_weights, (0, 2, 1))

    out = pl.pallas_call(
        functools.partial(_bl_kernel, L=L, NS=NS, K=K),
        grid=(L, NS),
        in_specs=[pl.BlockSpec((1, E, T), lambda l, s: (l, 0, s))],
        out_specs=pl.BlockSpec((1, 1), lambda l, s: (0, 0)),
        out_shape=jax.ShapeDtypeStruct((1, 1), jnp.float32),
        scratch_shapes=[
            pltpu.VMEM((L, E), jnp.float32),
            pltpu.VMEM((L, E), jnp.float32),
        ],
    )(rwt)

    scale = n_routed_experts / (S * num_experts_per_tok)
    return out[0, 0] * scale * (ALPHA / S)


# speedup vs baseline: 3.0691x; 1.4725x over previous
import functools

import jax
import jax.numpy as jnp
from jax.experimental import pallas as pl
from jax.experimental.pallas import tpu as pltpu

ALPHA = 0.01


def _bl_kernel(x_ref, loss_ref, counts_ref, sums_ref, *, L, NS, K):
    l = pl.program_id(0)
    s = pl.program_id(1)

    @pl.when(jnp.logical_and(l == 0, s == 0))
    def _init():
        counts_ref[...] = jnp.zeros_like(counts_ref)
        sums_ref[...] = jnp.zeros_like(sums_ref)

    T = x_ref.shape[2]
    TS = min(512, T)
    count_col = None
    sum_col = None
    for j in range(T // TS):
        x = x_ref[0, :, pl.ds(j * TS, TS)]
        sc = jnp.sum(x, axis=1, keepdims=True)
        thresh = jnp.max(x, axis=0, keepdims=True)
        for _ in range(K - 1):
            thresh = jnp.max(
                jnp.where(x < thresh, x, -jnp.inf), axis=0, keepdims=True
            )
        sel = (x >= thresh).astype(jnp.float32)
        cc = jnp.sum(sel, axis=1, keepdims=True)
        count_col = cc if count_col is None else count_col + cc
        sum_col = sc if sum_col is None else sum_col + sc
    sums_ref[pl.ds(l, 1), :] += sum_col.T
    counts_ref[pl.ds(l, 1), :] += count_col.T

    @pl.when(jnp.logical_and(l == L - 1, s == NS - 1))
    def _fin():
        loss_ref[...] = jnp.sum(counts_ref[...] * sums_ref[...]).reshape(1, 1)


def kernel(router_weights, n_routed_experts, num_experts_per_tok, router_n_groups):
    L, S, E = router_weights.shape
    K = 8
    T = min(16384, S)
    NS = S // T
    rwt = jnp.transpose(router_weights, (0, 2, 1))

    out = pl.pallas_call(
        functools.partial(_bl_kernel, L=L, NS=NS, K=K),
        grid=(L, NS),
        in_specs=[pl.BlockSpec((1, E, T), lambda l, s: (l, 0, s))],
        out_specs=pl.BlockSpec((1, 1), lambda l, s: (0, 0)),
        out_shape=jax.ShapeDtypeStruct((1, 1), jnp.float32),
        scratch_shapes=[
            pltpu.VMEM((L, E), jnp.float32),
            pltpu.VMEM((L, E), jnp.float32),
        ],
    )(rwt)

    scale = n_routed_experts / (S * num_experts_per_tok)
    return out[0, 0] * scale * (ALPHA / S)


# speedup vs baseline: 3.3025x; 1.0760x over previous
import functools

import jax
import jax.numpy as jnp
from jax.experimental import pallas as pl
from jax.experimental.pallas import tpu as pltpu

ALPHA = 0.01


def _bl_kernel(x_ref, loss_ref, counts_ref, sums_ref, *, L, NS, K):
    l = pl.program_id(0)
    s = pl.program_id(1)

    @pl.when(jnp.logical_and(l == 0, s == 0))
    def _init():
        counts_ref[...] = jnp.zeros_like(counts_ref)
        sums_ref[...] = jnp.zeros_like(sums_ref)

    T = x_ref.shape[2]
    TS = min(512, T)
    count_col = None
    sum_col = None
    for j in range(T // TS):
        x = x_ref[0, :, pl.ds(j * TS, TS)]
        sc = jnp.sum(x, axis=1, keepdims=True)
        thresh = None
        for r in range(K // 2):
            if thresh is None:
                xm = x
            else:
                xm = jnp.where(x < thresh, x, -jnp.inf)
            rows = [xm[g * 8 : (g + 1) * 8] for g in range(8)]
            ms = []
            for g in range(4):
                a, b = rows[2 * g], rows[2 * g + 1]
                ms.append((jnp.maximum(a, b), jnp.minimum(a, b)))
            while len(ms) > 1:
                nxt = []
                for g in range(len(ms) // 2):
                    (am, as_), (bm, bs) = ms[2 * g], ms[2 * g + 1]
                    nxt.append(
                        (
                            jnp.maximum(am, bm),
                            jnp.maximum(
                                jnp.minimum(am, bm), jnp.maximum(as_, bs)
                            ),
                        )
                    )
                ms = nxt
            m, s2 = ms[0]
            for off in (4, 2, 1):
                mr = pltpu.roll(m, off, 0)
                sr = pltpu.roll(s2, off, 0)
                s2 = jnp.maximum(jnp.minimum(m, mr), jnp.maximum(s2, sr))
                m = jnp.maximum(m, mr)
            thresh = s2[0:1]
        sel = (x >= thresh).astype(jnp.float32)
        cc = jnp.sum(sel, axis=1, keepdims=True)
        count_col = cc if count_col is None else count_col + cc
        sum_col = sc if sum_col is None else sum_col + sc
    sums_ref[pl.ds(l, 1), :] += sum_col.T
    counts_ref[pl.ds(l, 1), :] += count_col.T

    @pl.when(jnp.logical_and(l == L - 1, s == NS - 1))
    def _fin():
        loss_ref[...] = jnp.sum(counts_ref[...] * sums_ref[...]).reshape(1, 1)


def kernel(router_weights, n_routed_experts, num_experts_per_tok, router_n_groups):
    L, S, E = router_weights.shape
    K = 8
    T = min(16384, S)
    NS = S // T
    rwt = jnp.transpose(router_weights, (0, 2, 1))

    out = pl.pallas_call(
        functools.partial(_bl_kernel, L=L, NS=NS, K=K),
        grid=(L, NS),
        in_specs=[pl.BlockSpec((1, E, T), lambda l, s: (l, 0, s))],
        out_specs=pl.BlockSpec((1, 1), lambda l, s: (0, 0)),
        out_shape=jax.ShapeDtypeStruct((1, 1), jnp.float32),
        scratch_shapes=[
            pltpu.VMEM((L, E), jnp.float32),
            pltpu.VMEM((L, E), jnp.float32),
        ],
    )(rwt)

    scale = n_routed_experts / (S * num_experts_per_tok)
    return out[0, 0] * scale * (ALPHA / S)
